# baseline (speedup 1.0000x reference)
import functools

import jax
import jax.numpy as jnp
from jax import lax
from jax.experimental import pallas as pl
from jax.experimental.pallas import tpu as pltpu
from jax.experimental.pallas import tpu_sc as plsc

NC = 2
NS = 16
CH = 128


def _ceil_to(x, m):
    return (x + m - 1) // m * m


def _mesh():
    return plsc.VectorSubcoreMesh(core_axis_name="c", subcore_axis_name="s")


_SC_PARAMS = pltpu.CompilerParams(use_tc_tiling_on_sc=False)


def _sc_big(tlo, thi, idx, idx16, ec, zrow, z16, ones16, npad, stripe, c1):
    Hh = tlo.shape[1]
    f32 = jnp.float32
    c1h = c1
    out_type = (
        jax.ShapeDtypeStruct((npad, Hh), f32),
        jax.ShapeDtypeStruct((npad, Hh), f32),
        jax.ShapeDtypeStruct((npad, 16), f32),
        jax.ShapeDtypeStruct((npad, 16), f32),
    )
    scratch = [
        pltpu.VMEM((c1h, CH), jnp.int32),
        pltpu.VMEM((c1h, CH), jnp.int32),
        pltpu.VMEM((c1h, CH), jnp.int32),
        pltpu.VMEM((CH, Hh), f32),
        pltpu.VMEM((CH, 16), f32),
        pltpu.VMEM((CH, 16), f32),
        pltpu.VMEM_SHARED((npad, Hh), f32),
        pltpu.VMEM_SHARED((npad, 16), f32),
    ]

    @functools.partial(pl.kernel, mesh=_mesh(), out_type=out_type,
                       scratch_types=scratch, compiler_params=_SC_PARAMS)
    def k(tlo_hbm, thi_hbm, idx_hbm, idx16_hbm, ec_hbm,
          zrow_hbm, z16_hbm, ones_hbm,
          slo_o, shi_o, d_o, c_o,
          gi_v, si_v, ci_v, rows_v, ecr_v, ones_v, accS, acc16):
        cid = lax.axis_index("c")
        sid = lax.axis_index("s")
        sl = pl.ds(sid * stripe, stripe)
        pltpu.sync_copy(zrow_hbm, accS.at[sl])
        pltpu.sync_copy(z16_hbm, acc16.at[sl])
        pltpu.sync_copy(ones_hbm, ones_v)
        plsc.subcore_barrier()

        pltpu.sync_copy(idx_hbm.at[0, sid], gi_v)
        pltpu.sync_copy(idx_hbm.at[1, sid], si_v)
        pltpu.sync_copy(idx16_hbm.at[1, sid], ci_v)
        half = c1 // 2

        def loop_ones(lo, hi, tab):
            @pl.loop(lo, hi)
            def _(j):
                pltpu.sync_copy(tab.at[gi_v.at[j]], rows_v)
                pltpu.sync_copy(rows_v, accS.at[si_v.at[j]], add=True)
                pltpu.sync_copy(ones_v, acc16.at[si_v.at[j]], add=True)

        def loop_ec(lo, hi, tab):
            @pl.loop(lo, hi)
            def _(j):
                pltpu.sync_copy(tab.at[gi_v.at[j]], rows_v)
                pltpu.sync_copy(rows_v, accS.at[si_v.at[j]], add=True)
                pltpu.sync_copy(ec_hbm.at[sid, j], ecr_v)
                pltpu.sync_copy(ecr_v, acc16.at[ci_v.at[j]], add=True)

        @pl.when(cid == 0)
        def _():
            loop_ones(0, c1, tlo_hbm)

        @pl.when(cid == 1)
        def _():
            loop_ec(0, c1, thi_hbm)

        plsc.subcore_barrier()

        @pl.when(cid == 0)
        def _():
            pltpu.sync_copy(accS.at[sl], slo_o.at[sl])
            pltpu.sync_copy(acc16.at[sl], d_o.at[sl])

        @pl.when(cid == 1)
        def _():
            pltpu.sync_copy(accS.at[sl], shi_o.at[sl])
            pltpu.sync_copy(acc16.at[sl], c_o.at[sl])

    return k(tlo, thi, idx, idx16, ec, zrow, z16, ones16)


def _sc_quarter(tq0, tq1, idx, zrow, npad, stripe, c1):
    Hq = tq0.shape[1]
    f32 = jnp.float32
    scratch = [
        pltpu.VMEM((c1, CH), jnp.int32),
        pltpu.VMEM((c1, CH), jnp.int32),
        pltpu.VMEM((CH, Hq), f32),
        pltpu.VMEM_SHARED((npad, Hq), f32),
    ]

    @functools.partial(
        pl.kernel, mesh=_mesh(),
        out_type=(jax.ShapeDtypeStruct((npad, Hq), f32),
                  jax.ShapeDtypeStruct((npad, Hq), f32)),
        scratch_types=scratch, compiler_params=_SC_PARAMS)
    def k(tq0_hbm, tq1_hbm, idx_hbm, zrow_hbm, sq0_o, sq1_o,
          gi_v, si_v, rows_v, acc):
        cid = lax.axis_index("c")
        sid = lax.axis_index("s")
        sl = pl.ds(sid * stripe, stripe)
        pltpu.sync_copy(zrow_hbm, acc.at[sl])
        pltpu.sync_copy(idx_hbm.at[0, sid], gi_v)
        pltpu.sync_copy(idx_hbm.at[1, sid], si_v)
        plsc.subcore_barrier()

        def run(tab):
            @pl.loop(0, c1)
            def _(j):
                pltpu.sync_copy(tab.at[gi_v.at[j]], rows_v)
                pltpu.sync_copy(rows_v, acc.at[si_v.at[j]], add=True)

        @pl.when(cid == 0)
        def _():
            run(tq0_hbm)

        @pl.when(cid == 1)
        def _():
            run(tq1_hbm)

        plsc.subcore_barrier()

        @pl.when(cid == 0)
        def _():
            pltpu.sync_copy(acc.at[sl], sq0_o.at[sl])

        @pl.when(cid == 1)
        def _():
            pltpu.sync_copy(acc.at[sl], sq1_o.at[sl])

    return k(tq0, tq1, idx, zrow)


def _gru_half(S, D, h, WsT, WtT, b, WihT, WhhT, bih, bhh, H):
    d = D[:, 7:8]
    agg = jnp.dot(S, WsT, preferred_element_type=jnp.float32) + d * (
        jnp.dot(h, WtT, preferred_element_type=jnp.float32) + b)
    gi = jnp.dot(agg, WihT, preferred_element_type=jnp.float32) + bih
    gh = jnp.dot(h, WhhT, preferred_element_type=jnp.float32) + bhh
    r = jax.nn.sigmoid(gi[:, :H] + gh[:, :H])
    z = jax.nn.sigmoid(gi[:, H:2 * H] + gh[:, H:2 * H])
    n = jnp.tanh(gi[:, 2 * H:] + r * gh[:, 2 * H:])
    return (1.0 - z) * n + z * h


def _dense_gru(h, Slo, Shi, Da, Db, WsT, WtT, b, WihT, WhhT, bih, bhh, blk,
               emit_halves):
    N, H = h.shape
    Hh = H // 2
    grid = (N // blk,)
    row = pl.BlockSpec((blk, H), lambda i: (i, 0))
    rowh = pl.BlockSpec((blk, Hh), lambda i: (i, 0))
    row16 = pl.BlockSpec((blk, 16), lambda i: (i, 0))

    def w_spec(w):
        return pl.BlockSpec(w.shape, lambda i: tuple(0 for _ in w.shape))

    def body(h_r, Slo_r, Shi_r, Da_r, Db_r, Ws_r, Wt_r, b_r, Wih_r, Whh_r,
             bih_r, bhh_r, o_r, *halves):
        S = jnp.concatenate([Slo_r[...], Shi_r[...]], axis=1)
        D = Da_r[...] + Db_r[...]
        hg = _gru_half(S, D, h_r[...], Ws_r[...], Wt_r[...], b_r[...],
                       Wih_r[...], Whh_r[...], bih_r[...], bhh_r[...], H)
        o_r[...] = hg
        for i, q_r in enumerate(halves):
            q_r[...] = hg[:, i * Hh:(i + 1) * Hh]

    nq = 2 if emit_halves else 0
    ws = (WsT, WtT, b, WihT, WhhT, bih, bhh)
    res = pl.pallas_call(
        body,
        grid=grid,
        in_specs=[row, rowh, rowh, row16, row16] + [w_spec(w) for w in ws],
        out_specs=(row,) + (rowh,) * nq,
        out_shape=(jax.ShapeDtypeStruct((N, H), jnp.float32),)
                  + (jax.ShapeDtypeStruct((N, Hh), jnp.float32),) * nq,
    )(h, Slo, Shi, Da, Db, *ws)
    return res if emit_halves else res[0]


def _dense_couple(hb, Sq, Ca, Cb, AT, BT, M, b, blk, emit_halves):
    N, H = hb.shape
    Hh = H // 2
    grid = (N // blk,)
    row = pl.BlockSpec((blk, H), lambda i: (i, 0))
    rowh = pl.BlockSpec((blk, Hh), lambda i: (i, 0))
    row16 = pl.BlockSpec((blk, 16), lambda i: (i, 0))

    def w_spec(w):
        return pl.BlockSpec(w.shape, lambda i: tuple(0 for _ in w.shape))

    def body(hb_r, Slo_r, Shi_r, Ca_r, Cb_r, AT_r, BT_r, M_r, b_r,
             o_r, *halves):
        C_v = Ca_r[...] + Cb_r[...]
        d = C_v[:, 6:7]
        S = jnp.concatenate([Slo_r[...], Shi_r[...]], axis=1)
        o = (hb_r[...]
             + jnp.dot(S, AT_r[...], preferred_element_type=jnp.float32)
             + d * (jnp.dot(hb_r[...], BT_r[...],
                            preferred_element_type=jnp.float32) + b_r[...])
             + jnp.dot(C_v, M_r[...], preferred_element_type=jnp.float32))
        o_r[...] = o
        for i, q_r in enumerate(halves):
            q_r[...] = o[:, i * Hh:(i + 1) * Hh]

    nq = 2 if emit_halves else 0
    res = pl.pallas_call(
        body,
        grid=grid,
        in_specs=[row, rowh, rowh, row16, row16]
                 + [w_spec(w) for w in (AT, BT, M, b)],
        out_specs=(row,) + (rowh,) * nq,
        out_shape=(jax.ShapeDtypeStruct((N, H), jnp.float32),)
                  + (jax.ShapeDtypeStruct((N, Hh), jnp.float32),) * nq,
    )(hb, *Sq, Ca, Cb, AT, BT, M, b)
    return res if emit_halves else res[0]


def kernel(h1d, h2d, ei_1d, ei_2d, ei_coup, e_coup,
           W_msg1d, b_msg1d, W_msg2d, b_msg2d,
           W_c12, b_c12, W_c21, b_c21,
           Wih1, Whh1, bih1, bhh1, Wih2, Whh2, bih2, bhh2):
    N, H = h1d.shape
    Hh, Hq = H // 2, H // 4
    E = ei_1d.shape[1]
    F = e_coup.shape[1]
    f32 = jnp.float32

    stripe = _ceil_to((N + 1 + NS - 1) // NS, 8)
    npad = stripe * NS
    c1 = _ceil_to((E + NS * CH - 1) // (NS * CH), 2)
    ep = c1 * NS * CH

    def pad_idx(a, fill):
        return jnp.pad(a.astype(jnp.int32), (0, ep - E), constant_values=fill)

    def idx_pair(g, s):
        return jnp.stack([pad_idx(g, 0).reshape(NS, c1, CH),
                          pad_idx(s, N).reshape(NS, c1, CH)])

    idx1 = idx_pair(ei_1d[0], ei_1d[1])
    idx2 = idx_pair(ei_2d[0], ei_2d[1])
    idxc_fwd = idx_pair(ei_coup[0], ei_coup[1])
    idxc_rev = idx_pair(ei_coup[1], ei_coup[0])

    ec = jnp.zeros((E, 16), f32).at[:, :F].set(e_coup).at[:, F].set(1.0)
    ec = jnp.pad(ec, ((0, ep - E), (0, 0))).reshape(NS, c1, CH, 16)

    zrow = jnp.zeros((stripe, Hh), f32)
    zrowq = jnp.zeros((stripe, Hq), f32)
    z16 = jnp.zeros((stripe, 16), f32)
    ones16 = jnp.zeros((CH, 16), f32).at[:, 7].set(1.0)

    h1lo, h1hi = h1d[:, :Hh], h1d[:, Hh:]
    h2lo, h2hi = h2d[:, :Hh], h2d[:, Hh:]

    S1lo, S1hi, M1a, M1b = _sc_big(h1lo, h1hi, idx1, idxc_rev, ec, zrow, z16,
                                   ones16, npad, stripe, c1)
    S2lo, S2hi, M2a, M2b = _sc_big(h2lo, h2hi, idx2, idxc_fwd, ec, zrow, z16,
                                   ones16, npad, stripe, c1)

    blk = 1000 if N % 1000 == 0 else N
    h1g, h1glo, h1ghi = _dense_gru(
        h1d, S1lo, S1hi, M1a, M1b,
        W_msg1d[:, :H].T, W_msg1d[:, H:].T, b_msg1d.reshape(1, H),
        Wih1.T, Whh1.T, bih1.reshape(1, 3 * H), bhh1.reshape(1, 3 * H),
        blk, True)
    h2g = _dense_gru(
        h2d, S2lo, S2hi, M2a, M2b,
        W_msg2d[:, :H].T, W_msg2d[:, H:].T, b_msg2d.reshape(1, H),
        Wih2.T, Whh2.T, bih2.reshape(1, 3 * H), bhh2.reshape(1, 3 * H),
        blk, False)

    def pad_we(We):
        return jnp.zeros((16, H), f32).at[:F, :].set(We.T)

    Sc12lo, Sc12hi = _sc_quarter(h1glo, h1ghi, idxc_fwd, zrow, npad, stripe,
                                 c1)
    h2o, h2olo, h2ohi = _dense_couple(
        h2g, (Sc12lo, Sc12hi), M2a, M2b,
        W_c12[:, :H].T, W_c12[:, H:2 * H].T,
        pad_we(W_c12[:, 2 * H:]), b_c12.reshape(1, H), blk, True)

    Sc21lo, Sc21hi = _sc_quarter(h2olo, h2ohi, idxc_rev, zrow, npad, stripe,
                                 c1)
    h1o = _dense_couple(
        h1g, (Sc21lo, Sc21hi), M1a, M1b,
        W_c21[:, :H].T, W_c21[:, H:2 * H].T,
        pad_we(W_c21[:, 2 * H:]), b_c21.reshape(1, H), blk, False)

    return (h1o, h2o)


# speedup vs baseline: 1.1574x; 1.1574x over previous
import functools

import jax
import jax.numpy as jnp
from jax import lax
from jax.experimental import pallas as pl
from jax.experimental.pallas import tpu as pltpu
from jax.experimental.pallas import tpu_sc as plsc

NC = 2
NS = 16
CH = 128


def _ceil_to(x, m):
    return (x + m - 1) // m * m


def _mesh():
    return plsc.VectorSubcoreMesh(core_axis_name="c", subcore_axis_name="s")


_SC_PARAMS = pltpu.CompilerParams(use_tc_tiling_on_sc=False)


def _sc_big(tlo, thi, idx, idx16, ec, zrow, z16, ones16, npad, stripe, c1):
    Hh = tlo.shape[1]
    f32 = jnp.float32
    c1h = c1
    out_type = (
        jax.ShapeDtypeStruct((npad, Hh), f32),
        jax.ShapeDtypeStruct((npad, Hh), f32),
        jax.ShapeDtypeStruct((npad, 16), f32),
        jax.ShapeDtypeStruct((npad, 16), f32),
    )
    scratch = [
        pltpu.VMEM((c1h, CH), jnp.int32),
        pltpu.VMEM((c1h, CH), jnp.int32),
        pltpu.VMEM((c1h, CH), jnp.int32),
        pltpu.VMEM((CH, Hh), f32),
        pltpu.VMEM((CH, 16), f32),
        pltpu.VMEM((CH, 16), f32),
        pltpu.VMEM_SHARED((npad, Hh), f32),
        pltpu.VMEM_SHARED((npad, 16), f32),
    ]

    @functools.partial(pl.kernel, mesh=_mesh(), out_type=out_type,
                       scratch_types=scratch, compiler_params=_SC_PARAMS)
    def k(tlo_hbm, thi_hbm, idx_hbm, idx16_hbm, ec_hbm,
          zrow_hbm, z16_hbm, ones_hbm,
          slo_o, shi_o, d_o, c_o,
          gi_v, si_v, ci_v, rows_v, ecr_v, ones_v, accS, acc16):
        cid = lax.axis_index("c")
        sid = lax.axis_index("s")
        sl = pl.ds(sid * stripe, stripe)
        pltpu.sync_copy(zrow_hbm, accS.at[sl])
        pltpu.sync_copy(z16_hbm, acc16.at[sl])
        pltpu.sync_copy(ones_hbm, ones_v)
        plsc.subcore_barrier()

        pltpu.sync_copy(idx_hbm.at[0, sid], gi_v)
        pltpu.sync_copy(idx_hbm.at[1, sid], si_v)
        pltpu.sync_copy(idx16_hbm.at[1, sid], ci_v)
        plsc.subcore_barrier()

        @pl.when(cid == 0)
        def _():
            @pl.loop(0, c1)
            def _(j):
                pltpu.sync_copy(tlo_hbm.at[gi_v.at[j]], rows_v)
                pltpu.sync_copy(rows_v, accS.at[si_v.at[j]], add=True)
                pltpu.sync_copy(ones_v, acc16.at[si_v.at[j]], add=True)

        @pl.when(cid == 1)
        def _():
            @pl.loop(0, c1)
            def _(j):
                pltpu.sync_copy(thi_hbm.at[gi_v.at[j]], rows_v)
                pltpu.sync_copy(rows_v, accS.at[si_v.at[j]], add=True)
                pltpu.sync_copy(ec_hbm.at[sid, j], ecr_v)
                pltpu.sync_copy(ecr_v, acc16.at[ci_v.at[j]], add=True)

        plsc.subcore_barrier()

        @pl.when(cid == 0)
        def _():
            pltpu.sync_copy(accS.at[sl], slo_o.at[sl])
            pltpu.sync_copy(acc16.at[sl], d_o.at[sl])

        @pl.when(cid == 1)
        def _():
            pltpu.sync_copy(accS.at[sl], shi_o.at[sl])
            pltpu.sync_copy(acc16.at[sl], c_o.at[sl])

    return k(tlo, thi, idx, idx16, ec, zrow, z16, ones16)


def _sc_quarter(tq0, tq1, idx, zrow, npad, stripe, c1):
    Hq = tq0.shape[1]
    f32 = jnp.float32
    scratch = [
        pltpu.VMEM((c1, CH), jnp.int32),
        pltpu.VMEM((c1, CH), jnp.int32),
        pltpu.VMEM((CH, Hq), f32),
        pltpu.VMEM_SHARED((npad, Hq), f32),
    ]

    @functools.partial(
        pl.kernel, mesh=_mesh(),
        out_type=(jax.ShapeDtypeStruct((npad, Hq), f32),
                  jax.ShapeDtypeStruct((npad, Hq), f32)),
        scratch_types=scratch, compiler_params=_SC_PARAMS)
    def k(tq0_hbm, tq1_hbm, idx_hbm, zrow_hbm, sq0_o, sq1_o,
          gi_v, si_v, rows_v, acc):
        cid = lax.axis_index("c")
        sid = lax.axis_index("s")
        sl = pl.ds(sid * stripe, stripe)
        pltpu.sync_copy(zrow_hbm, acc.at[sl])
        pltpu.sync_copy(idx_hbm.at[0, sid], gi_v)
        pltpu.sync_copy(idx_hbm.at[1, sid], si_v)
        plsc.subcore_barrier()

        def run(tab):
            @pl.loop(0, c1)
            def _(j):
                pltpu.sync_copy(tab.at[gi_v.at[j]], rows_v)
                pltpu.sync_copy(rows_v, acc.at[si_v.at[j]], add=True)

        @pl.when(cid == 0)
        def _():
            run(tq0_hbm)

        @pl.when(cid == 1)
        def _():
            run(tq1_hbm)

        plsc.subcore_barrier()

        @pl.when(cid == 0)
        def _():
            pltpu.sync_copy(acc.at[sl], sq0_o.at[sl])

        @pl.when(cid == 1)
        def _():
            pltpu.sync_copy(acc.at[sl], sq1_o.at[sl])

    return k(tq0, tq1, idx, zrow)


def _gru_half(S, D, h, WsT, WtT, b, WihT, WhhT, bih, bhh, H):
    d = D[:, 0:1]
    agg = jnp.dot(S, WsT, preferred_element_type=jnp.float32) + d * (
        jnp.dot(h, WtT, preferred_element_type=jnp.float32) + b)
    gi = jnp.dot(agg, WihT, preferred_element_type=jnp.float32) + bih
    gh = jnp.dot(h, WhhT, preferred_element_type=jnp.float32) + bhh
    r = jax.nn.sigmoid(gi[:, :H] + gh[:, :H])
    z = jax.nn.sigmoid(gi[:, H:2 * H] + gh[:, H:2 * H])
    n = jnp.tanh(gi[:, 2 * H:] + r * gh[:, 2 * H:])
    return (1.0 - z) * n + z * h


def _dense_gru(h, Slo, Shi, D, WsT, WtT, b, WihT, WhhT, bih, bhh, blk,
               emit_halves):
    N, H = h.shape
    Hh = H // 2
    grid = (N // blk,)
    row = pl.BlockSpec((blk, H), lambda i: (i, 0))
    rowh = pl.BlockSpec((blk, Hh), lambda i: (i, 0))
    row16 = pl.BlockSpec((blk, 16), lambda i: (i, 0))

    def w_spec(w):
        return pl.BlockSpec(w.shape, lambda i: tuple(0 for _ in w.shape))

    def body(h_r, Slo_r, Shi_r, D_r, Ws_r, Wt_r, b_r, Wih_r, Whh_r,
             bih_r, bhh_r, o_r, *halves):
        S = jnp.concatenate([Slo_r[...], Shi_r[...]], axis=1)
        hg = _gru_half(S, D_r[...], h_r[...], Ws_r[...], Wt_r[...], b_r[...],
                       Wih_r[...], Whh_r[...], bih_r[...], bhh_r[...], H)
        o_r[...] = hg
        for i, q_r in enumerate(halves):
            q_r[...] = hg[:, i * Hh:(i + 1) * Hh]

    nq = 2 if emit_halves else 0
    ws = (WsT, WtT, b, WihT, WhhT, bih, bhh)
    res = pl.pallas_call(
        body,
        grid=grid,
        in_specs=[row, rowh, rowh, row16] + [w_spec(w) for w in ws],
        out_specs=(row,) + (rowh,) * nq,
        out_shape=(jax.ShapeDtypeStruct((N, H), jnp.float32),)
                  + (jax.ShapeDtypeStruct((N, Hh), jnp.float32),) * nq,
    )(h, Slo, Shi, D, *ws)
    return res if emit_halves else res[0]


def _dense_couple(hb, Sq, C, AT, BT, M, b, blk, emit_halves):
    N, H = hb.shape
    Hh = H // 2
    grid = (N // blk,)
    row = pl.BlockSpec((blk, H), lambda i: (i, 0))
    rowh = pl.BlockSpec((blk, Hh), lambda i: (i, 0))
    row16 = pl.BlockSpec((blk, 16), lambda i: (i, 0))

    def w_spec(w):
        return pl.BlockSpec(w.shape, lambda i: tuple(0 for _ in w.shape))

    def body(hb_r, Slo_r, Shi_r, C_r, AT_r, BT_r, M_r, b_r,
             o_r, *halves):
        C_v = C_r[...]
        d = C_v[:, 6:7]
        S = jnp.concatenate([Slo_r[...], Shi_r[...]], axis=1)
        o = (hb_r[...]
             + jnp.dot(S, AT_r[...], preferred_element_type=jnp.float32)
             + d * (jnp.dot(hb_r[...], BT_r[...],
                            preferred_element_type=jnp.float32) + b_r[...])
             + jnp.dot(C_v, M_r[...], preferred_element_type=jnp.float32))
        o_r[...] = o
        for i, q_r in enumerate(halves):
            q_r[...] = o[:, i * Hh:(i + 1) * Hh]

    nq = 2 if emit_halves else 0
    res = pl.pallas_call(
        body,
        grid=grid,
        in_specs=[row, rowh, rowh, row16]
                 + [w_spec(w) for w in (AT, BT, M, b)],
        out_specs=(row,) + (rowh,) * nq,
        out_shape=(jax.ShapeDtypeStruct((N, H), jnp.float32),)
                  + (jax.ShapeDtypeStruct((N, Hh), jnp.float32),) * nq,
    )(hb, *Sq, C, AT, BT, M, b)
    return res if emit_halves else res[0]


def kernel(h1d, h2d, ei_1d, ei_2d, ei_coup, e_coup,
           W_msg1d, b_msg1d, W_msg2d, b_msg2d,
           W_c12, b_c12, W_c21, b_c21,
           Wih1, Whh1, bih1, bhh1, Wih2, Whh2, bih2, bhh2):
    N, H = h1d.shape
    Hh, Hq = H // 2, H // 4
    E = ei_1d.shape[1]
    F = e_coup.shape[1]
    f32 = jnp.float32

    stripe = _ceil_to((N + 1 + NS - 1) // NS, 8)
    npad = stripe * NS
    c1 = (E + NS * CH - 1) // (NS * CH)
    ep = c1 * NS * CH

    def pad_idx(a, fill):
        return jnp.pad(a.astype(jnp.int32), (0, ep - E), constant_values=fill)

    def idx_pair(g, s):
        return jnp.stack([pad_idx(g, 0).reshape(NS, c1, CH),
                          pad_idx(s, N).reshape(NS, c1, CH)])

    idx1 = idx_pair(ei_1d[0], ei_1d[1])
    idx2 = idx_pair(ei_2d[0], ei_2d[1])
    idxc_fwd = idx_pair(ei_coup[0], ei_coup[1])
    idxc_rev = idx_pair(ei_coup[1], ei_coup[0])

    ec = jnp.zeros((E, 16), f32).at[:, :F].set(e_coup).at[:, F].set(1.0)
    ec = jnp.pad(ec, ((0, ep - E), (0, 0))).reshape(NS, c1, CH, 16)

    zrow = jnp.zeros((stripe, Hh), f32)
    zrowq = jnp.zeros((stripe, Hq), f32)
    z16 = jnp.zeros((stripe, 16), f32)
    ones16 = jnp.ones((CH, 16), f32)

    h1lo, h1hi = h1d[:, :Hh], h1d[:, Hh:]
    h2lo, h2hi = h2d[:, :Hh], h2d[:, Hh:]

    S1lo, S1hi, D1p, C1p = _sc_big(h1lo, h1hi, idx1, idxc_rev, ec, zrow, z16,
                                   ones16, npad, stripe, c1)
    S2lo, S2hi, D2p, C2p = _sc_big(h2lo, h2hi, idx2, idxc_fwd, ec, zrow, z16,
                                   ones16, npad, stripe, c1)

    blk = 1000 if N % 1000 == 0 else N
    h1g, h1glo, h1ghi = _dense_gru(
        h1d, S1lo, S1hi, D1p,
        W_msg1d[:, :H].T, W_msg1d[:, H:].T, b_msg1d.reshape(1, H),
        Wih1.T, Whh1.T, bih1.reshape(1, 3 * H), bhh1.reshape(1, 3 * H),
        blk, True)
    h2g = _dense_gru(
        h2d, S2lo, S2hi, D2p,
        W_msg2d[:, :H].T, W_msg2d[:, H:].T, b_msg2d.reshape(1, H),
        Wih2.T, Whh2.T, bih2.reshape(1, 3 * H), bhh2.reshape(1, 3 * H),
        blk, False)

    def pad_we(We):
        return jnp.zeros((16, H), f32).at[:F, :].set(We.T)

    Sc12lo, Sc12hi = _sc_quarter(h1glo, h1ghi, idxc_fwd, zrow, npad, stripe,
                                 c1)
    h2o, h2olo, h2ohi = _dense_couple(
        h2g, (Sc12lo, Sc12hi), C2p,
        W_c12[:, :H].T, W_c12[:, H:2 * H].T,
        pad_we(W_c12[:, 2 * H:]), b_c12.reshape(1, H), blk, True)

    Sc21lo, Sc21hi = _sc_quarter(h2olo, h2ohi, idxc_rev, zrow, npad, stripe,
                                 c1)
    h1o = _dense_couple(
        h1g, (Sc21lo, Sc21hi), C1p,
        W_c21[:, :H].T, W_c21[:, H:2 * H].T,
        pad_we(W_c21[:, 2 * H:]), b_c21.reshape(1, H), blk, False)

    return (h1o, h2o)


# speedup vs baseline: 1.1588x; 1.0012x over previous
import functools

import jax
import jax.numpy as jnp
from jax import lax
from jax.experimental import pallas as pl
from jax.experimental.pallas import tpu as pltpu
from jax.experimental.pallas import tpu_sc as plsc

NC = 2
NS = 16
CH = 128


def _ceil_to(x, m):
    return (x + m - 1) // m * m


def _mesh():
    return plsc.VectorSubcoreMesh(core_axis_name="c", subcore_axis_name="s")


_SC_PARAMS = pltpu.CompilerParams(use_tc_tiling_on_sc=False)


def _sc_big(tlo, thi, idx, idx16, ec, zrow, z16, ones16, npad, stripe, c1):
    Hh = tlo.shape[1]
    f32 = jnp.float32
    c1h = c1
    out_type = (
        jax.ShapeDtypeStruct((npad, Hh), f32),
        jax.ShapeDtypeStruct((npad, Hh), f32),
        jax.ShapeDtypeStruct((npad, 16), f32),
        jax.ShapeDtypeStruct((npad, 16), f32),
    )
    scratch = [
        pltpu.VMEM((c1h, CH), jnp.int32),
        pltpu.VMEM((c1h, CH), jnp.int32),
        pltpu.VMEM((c1h, CH), jnp.int32),
        pltpu.VMEM((CH, Hh), f32),
        pltpu.VMEM((CH, 16), f32),
        pltpu.VMEM((CH, 16), f32),
        pltpu.VMEM_SHARED((npad, Hh), f32),
        pltpu.VMEM_SHARED((npad, 16), f32),
    ]

    @functools.partial(pl.kernel, mesh=_mesh(), out_type=out_type,
                       scratch_types=scratch, compiler_params=_SC_PARAMS)
    def k(tlo_hbm, thi_hbm, idx_hbm, idx16_hbm, ec_hbm,
          zrow_hbm, z16_hbm, ones_hbm,
          slo_o, shi_o, d_o, c_o,
          gi_v, si_v, ci_v, rows_v, ecr_v, ones_v, accS, acc16):
        cid = lax.axis_index("c")
        sid = lax.axis_index("s")
        sl = pl.ds(sid * stripe, stripe)
        pltpu.sync_copy(zrow_hbm, accS.at[sl])
        pltpu.sync_copy(z16_hbm, acc16.at[sl])
        pltpu.sync_copy(ones_hbm, ones_v)
        plsc.subcore_barrier()

        pltpu.sync_copy(idx_hbm.at[0, sid], gi_v)
        pltpu.sync_copy(idx_hbm.at[1, sid], si_v)
        pltpu.sync_copy(idx16_hbm.at[1, sid], ci_v)
        plsc.subcore_barrier()

        @pl.when(cid == 0)
        def _():
            @pl.loop(0, c1)
            def _(j):
                pltpu.sync_copy(tlo_hbm.at[gi_v.at[j]], rows_v)
                pltpu.sync_copy(rows_v, accS.at[si_v.at[j]], add=True)
                pltpu.sync_copy(ones_v, acc16.at[si_v.at[j]], add=True)

        @pl.when(cid == 1)
        def _():
            @pl.loop(0, c1)
            def _(j):
                pltpu.sync_copy(thi_hbm.at[gi_v.at[j]], rows_v)
                pltpu.sync_copy(rows_v, accS.at[si_v.at[j]], add=True)
                pltpu.sync_copy(ec_hbm.at[sid, j], ecr_v)
                pltpu.sync_copy(ecr_v, acc16.at[ci_v.at[j]], add=True)

        plsc.subcore_barrier()

        @pl.when(cid == 0)
        def _():
            pltpu.sync_copy(accS.at[sl], slo_o.at[sl])
            pltpu.sync_copy(acc16.at[sl], d_o.at[sl])

        @pl.when(cid == 1)
        def _():
            pltpu.sync_copy(accS.at[sl], shi_o.at[sl])
            pltpu.sync_copy(acc16.at[sl], c_o.at[sl])

    return k(tlo, thi, idx, idx16, ec, zrow, z16, ones16)


def _sc_half(tq0, tq1, idx, zrow, npad, stripe, c1):
    Hq = tq0.shape[1]
    f32 = jnp.float32
    scratch = [
        pltpu.VMEM((c1, CH), jnp.int32),
        pltpu.VMEM((c1, CH), jnp.int32),
        pltpu.VMEM((CH, Hq), f32),
        pltpu.VMEM_SHARED((npad, Hq), f32),
    ]

    @functools.partial(
        pl.kernel, mesh=_mesh(),
        out_type=(jax.ShapeDtypeStruct((npad, Hq), f32),
                  jax.ShapeDtypeStruct((npad, Hq), f32)),
        scratch_types=scratch, compiler_params=_SC_PARAMS)
    def k(tq0_hbm, tq1_hbm, idx_hbm, zrow_hbm, sq0_o, sq1_o,
          gi_v, si_v, rows_v, acc):
        cid = lax.axis_index("c")
        sid = lax.axis_index("s")
        sl = pl.ds(sid * stripe, stripe)
        pltpu.sync_copy(zrow_hbm, acc.at[sl])
        pltpu.sync_copy(idx_hbm.at[0, sid], gi_v)
        pltpu.sync_copy(idx_hbm.at[1, sid], si_v)
        plsc.subcore_barrier()

        def run(tab):
            @pl.loop(0, c1)
            def _(j):
                pltpu.sync_copy(tab.at[gi_v.at[j]], rows_v)
                pltpu.sync_copy(rows_v, acc.at[si_v.at[j]], add=True)

        @pl.when(cid == 0)
        def _():
            run(tq0_hbm)

        @pl.when(cid == 1)
        def _():
            run(tq1_hbm)

        plsc.subcore_barrier()

        @pl.when(cid == 0)
        def _():
            pltpu.sync_copy(acc.at[sl], sq0_o.at[sl])

        @pl.when(cid == 1)
        def _():
            pltpu.sync_copy(acc.at[sl], sq1_o.at[sl])

    return k(tq0, tq1, idx, zrow)


def _gru_half(S, D, h, WsT, WtT, b, WihT, WhhT, bih, bhh, H):
    d = D[:, 0:1]
    agg = jnp.dot(S, WsT, preferred_element_type=jnp.float32) + d * (
        jnp.dot(h, WtT, preferred_element_type=jnp.float32) + b)
    gi = jnp.dot(agg, WihT, preferred_element_type=jnp.float32) + bih
    gh = jnp.dot(h, WhhT, preferred_element_type=jnp.float32) + bhh
    r = jax.nn.sigmoid(gi[:, :H] + gh[:, :H])
    z = jax.nn.sigmoid(gi[:, H:2 * H] + gh[:, H:2 * H])
    n = jnp.tanh(gi[:, 2 * H:] + r * gh[:, 2 * H:])
    return (1.0 - z) * n + z * h


def _dense_gru(h, Slo, Shi, D, WsT, WtT, b, WihT, WhhT, bih, bhh, blk,
               emit_halves):
    N, H = h.shape
    Hh = H // 2
    grid = (N // blk,)
    row = pl.BlockSpec((blk, H), lambda i: (i, 0))
    rowh = pl.BlockSpec((blk, Hh), lambda i: (i, 0))
    row16 = pl.BlockSpec((blk, 16), lambda i: (i, 0))

    def w_spec(w):
        return pl.BlockSpec(w.shape, lambda i: tuple(0 for _ in w.shape))

    def body(h_r, Slo_r, Shi_r, D_r, Ws_r, Wt_r, b_r, Wih_r, Whh_r,
             bih_r, bhh_r, o_r, *halves):
        S = jnp.concatenate([Slo_r[...], Shi_r[...]], axis=1)
        hg = _gru_half(S, D_r[...], h_r[...], Ws_r[...], Wt_r[...], b_r[...],
                       Wih_r[...], Whh_r[...], bih_r[...], bhh_r[...], H)
        o_r[...] = hg
        for i, q_r in enumerate(halves):
            q_r[...] = hg[:, i * Hh:(i + 1) * Hh]

    nq = 2 if emit_halves else 0
    ws = (WsT, WtT, b, WihT, WhhT, bih, bhh)
    res = pl.pallas_call(
        body,
        grid=grid,
        in_specs=[row, rowh, rowh, row16] + [w_spec(w) for w in ws],
        out_specs=(row,) + (rowh,) * nq,
        out_shape=(jax.ShapeDtypeStruct((N, H), jnp.float32),)
                  + (jax.ShapeDtypeStruct((N, Hh), jnp.float32),) * nq,
    )(h, Slo, Shi, D, *ws)
    return res if emit_halves else res[0]


def _dense_couple(hb, Sq, C, AT, BT, M, b, blk, emit_halves):
    N, H = hb.shape
    Hh = H // 2
    grid = (N // blk,)
    row = pl.BlockSpec((blk, H), lambda i: (i, 0))
    rowh = pl.BlockSpec((blk, Hh), lambda i: (i, 0))
    row16 = pl.BlockSpec((blk, 16), lambda i: (i, 0))

    def w_spec(w):
        return pl.BlockSpec(w.shape, lambda i: tuple(0 for _ in w.shape))

    def body(hb_r, Slo_r, Shi_r, C_r, AT_r, BT_r, M_r, b_r,
             o_r, *halves):
        C_v = C_r[...]
        d = C_v[:, 6:7]
        S = jnp.concatenate([Slo_r[...], Shi_r[...]], axis=1)
        o = (hb_r[...]
             + jnp.dot(S, AT_r[...], preferred_element_type=jnp.float32)
             + d * (jnp.dot(hb_r[...], BT_r[...],
                            preferred_element_type=jnp.float32) + b_r[...])
             + jnp.dot(C_v, M_r[...], preferred_element_type=jnp.float32))
        o_r[...] = o
        for i, q_r in enumerate(halves):
            q_r[...] = o[:, i * Hh:(i + 1) * Hh]

    nq = 2 if emit_halves else 0
    res = pl.pallas_call(
        body,
        grid=grid,
        in_specs=[row, rowh, rowh, row16]
                 + [w_spec(w) for w in (AT, BT, M, b)],
        out_specs=(row,) + (rowh,) * nq,
        out_shape=(jax.ShapeDtypeStruct((N, H), jnp.float32),)
                  + (jax.ShapeDtypeStruct((N, Hh), jnp.float32),) * nq,
    )(hb, *Sq, C, AT, BT, M, b)
    return res if emit_halves else res[0]


def kernel(h1d, h2d, ei_1d, ei_2d, ei_coup, e_coup,
           W_msg1d, b_msg1d, W_msg2d, b_msg2d,
           W_c12, b_c12, W_c21, b_c21,
           Wih1, Whh1, bih1, bhh1, Wih2, Whh2, bih2, bhh2):
    N, H = h1d.shape
    Hh, Hq = H // 2, H // 4
    E = ei_1d.shape[1]
    F = e_coup.shape[1]
    f32 = jnp.float32

    stripe = _ceil_to((N + 1 + NS - 1) // NS, 8)
    npad = stripe * NS
    c1 = (E + NS * CH - 1) // (NS * CH)
    ep = c1 * NS * CH

    def pad_idx(a, fill):
        return jnp.pad(a.astype(jnp.int32), (0, ep - E), constant_values=fill)

    def idx_pair(g, s):
        return jnp.stack([pad_idx(g, 0).reshape(NS, c1, CH),
                          pad_idx(s, N).reshape(NS, c1, CH)])

    idx1 = idx_pair(ei_1d[0], ei_1d[1])
    idx2 = idx_pair(ei_2d[0], ei_2d[1])
    idxc_fwd = idx_pair(ei_coup[0], ei_coup[1])
    idxc_rev = idx_pair(ei_coup[1], ei_coup[0])

    ec = jnp.zeros((E, 16), f32).at[:, :F].set(e_coup).at[:, F].set(1.0)
    ec = jnp.pad(ec, ((0, ep - E), (0, 0))).reshape(NS, c1, CH, 16)

    zrow = jnp.zeros((stripe, Hh), f32)
    zrowq = jnp.zeros((stripe, Hq), f32)
    z16 = jnp.zeros((stripe, 16), f32)
    ones16 = jnp.ones((CH, 16), f32)

    h1lo, h1hi = h1d[:, :Hh], h1d[:, Hh:]
    h2lo, h2hi = h2d[:, :Hh], h2d[:, Hh:]

    S1lo, S1hi, D1p, C1p = _sc_big(h1lo, h1hi, idx1, idxc_rev, ec, zrow, z16,
                                   ones16, npad, stripe, c1)
    S2lo, S2hi, D2p, C2p = _sc_big(h2lo, h2hi, idx2, idxc_fwd, ec, zrow, z16,
                                   ones16, npad, stripe, c1)

    blk = 1000 if N % 1000 == 0 else N
    h1g, h1glo, h1ghi = _dense_gru(
        h1d, S1lo, S1hi, D1p,
        W_msg1d[:, :H].T, W_msg1d[:, H:].T, b_msg1d.reshape(1, H),
        Wih1.T, Whh1.T, bih1.reshape(1, 3 * H), bhh1.reshape(1, 3 * H),
        blk, True)
    h2g = _dense_gru(
        h2d, S2lo, S2hi, D2p,
        W_msg2d[:, :H].T, W_msg2d[:, H:].T, b_msg2d.reshape(1, H),
        Wih2.T, Whh2.T, bih2.reshape(1, 3 * H), bhh2.reshape(1, 3 * H),
        blk, False)

    def pad_we(We):
        return jnp.zeros((16, H), f32).at[:F, :].set(We.T)

    Sc12lo, Sc12hi = _sc_half(h1glo, h1ghi, idxc_fwd, zrow, npad, stripe,
                                 c1)
    h2o, h2olo, h2ohi = _dense_couple(
        h2g, (Sc12lo, Sc12hi), C2p,
        W_c12[:, :H].T, W_c12[:, H:2 * H].T,
        pad_we(W_c12[:, 2 * H:]), b_c12.reshape(1, H), blk, True)

    Sc21lo, Sc21hi = _sc_half(h2olo, h2ohi, idxc_rev, zrow, npad, stripe,
                                 c1)
    h1o = _dense_couple(
        h1g, (Sc21lo, Sc21hi), C1p,
        W_c21[:, :H].T, W_c21[:, H:2 * H].T,
        pad_we(W_c21[:, 2 * H:]), b_c21.reshape(1, H), blk, False)

    return (h1o, h2o)
